# baseline (speedup 1.0000x reference)
import jax
import jax.numpy as jnp
import numpy as np
from jax import lax
from jax.experimental import pallas as pl
from jax.experimental.pallas import tpu as pltpu
from jax.experimental.pallas import tpu_sc as plsc

N = 10000
E = 320000
H = 128
EPAD = 327680
EROWS = EPAD // H
RW = 80
ACC = 10240
STRIPE = ACC // 16
BM = 2000

_f32 = jnp.float32


def _deg_body(dst_hbm, deg_out, acc3, dst_v, ones_v, zero_v):
    c = lax.axis_index("c")
    s = lax.axis_index("s")
    wid = c * 16 + s

    def z640(j, carry):
        zero_v[pl.ds(j * 16, 16)] = jnp.zeros((16,), _f32)
        return carry
    lax.fori_loop(0, 40, z640, 0)
    pltpu.sync_copy(zero_v, acc3.at[pl.ds(s * 640, 640)])

    def o128(j, carry):
        ones_v[pl.ds(j * 16, 16)] = jnp.full((16,), 1.0, _f32)
        return carry
    lax.fori_loop(0, 8, o128, 0)

    pltpu.sync_copy(dst_hbm.at[pl.ds(wid * RW, RW)], dst_v)
    plsc.subcore_barrier()

    def scat(i, carry):
        pltpu.sync_copy(ones_v, acc3.at[dst_v.at[i]], add=True)
        return carry
    lax.fori_loop(0, RW, scat, 0)
    plsc.subcore_barrier()

    pltpu.sync_copy(acc3.at[pl.ds(s * 640, 640)],
                    deg_out.at[pl.ds(c * ACC + s * 640, 640)])


_deg_call = pl.kernel(
    _deg_body,
    out_type=jax.ShapeDtypeStruct((2 * ACC,), _f32),
    mesh=plsc.VectorSubcoreMesh(core_axis_name="c", subcore_axis_name="s"),
    scratch_types=[
        pltpu.VMEM_SHARED((ACC,), _f32),
        pltpu.VMEM((RW, 128), jnp.int32),
        pltpu.VMEM((128,), _f32),
        pltpu.VMEM((640,), _f32),
    ],
)


def _agg_body(g_hbm, src_hbm, dst_hbm, p_out, acc, src_v, dst_v,
              r0, s0, s1, s2, s3):
    c = lax.axis_index("c")
    s = lax.axis_index("s")
    wid = c * 16 + s

    def fill_zero(k, carry):
        i = k // 8
        j = lax.rem(k, 8)
        r0[i, pl.ds(j * 16, 16)] = jnp.zeros((16,), _f32)
        return carry
    lax.fori_loop(0, 1024, fill_zero, 0)

    for k in range(STRIPE // 128):
        pltpu.sync_copy(r0, acc.at[pl.ds(s * STRIPE + k * 128, 128)])
    plsc.subcore_barrier()

    pltpu.sync_copy(src_hbm.at[pl.ds(wid * RW, RW)], src_v.at[pl.ds(0, RW)])
    pltpu.sync_copy(src_hbm.at[pl.ds(wid * RW, 8)], src_v.at[pl.ds(RW, 8)])
    pltpu.sync_copy(dst_hbm.at[pl.ds(wid * RW, RW)], dst_v)

    quarters = [(r0.at[pl.ds(32 * q, 32)], 32 * q) for q in range(4)]
    sems = [s0, s1, s2, s3]
    for (hq, off), sq in zip(quarters, sems):
        pltpu.async_copy(g_hbm.at[src_v.at[0, pl.ds(off, 32)]], hq, sq)

    def step(i, carry):
        for (hq, off), sq in zip(quarters, sems):
            pltpu.make_async_copy(g_hbm.at[src_v.at[i, pl.ds(off, 32)]],
                                  hq, sq).wait()
        pltpu.sync_copy(r0, acc.at[dst_v.at[i]], add=True)
        for (hq, off), sq in zip(quarters, sems):
            pltpu.async_copy(g_hbm.at[src_v.at[i + 1, pl.ds(off, 32)]], hq, sq)
        return carry
    lax.fori_loop(0, RW, step, 0)
    for (hq, off), sq in zip(quarters, sems):
        pltpu.make_async_copy(g_hbm.at[src_v.at[RW, pl.ds(off, 32)]],
                              hq, sq).wait()
    plsc.subcore_barrier()

    pltpu.sync_copy(acc.at[pl.ds(s * STRIPE, STRIPE)],
                    p_out.at[c, pl.ds(s * STRIPE, STRIPE)])


_agg_call = pl.kernel(
    _agg_body,
    out_type=jax.ShapeDtypeStruct((2, ACC, H), _f32),
    mesh=plsc.VectorSubcoreMesh(core_axis_name="c", subcore_axis_name="s"),
    scratch_types=[
        pltpu.VMEM_SHARED((ACC, H), _f32),
        pltpu.VMEM((RW + 8, 128), jnp.int32),
        pltpu.VMEM((RW, 128), jnp.int32),
        pltpu.VMEM((128, H), _f32),
        pltpu.SemaphoreType.DMA,
        pltpu.SemaphoreType.DMA,
        pltpu.SemaphoreType.DMA,
        pltpu.SemaphoreType.DMA,
    ],
)


def _dinv(d0_ref, d1_ref):
    return lax.rsqrt(d0_ref[...] + d1_ref[...] + 1.0)


def _embed_body(x_ref, we_ref, be_ref, w1_ref, d0_ref, d1_ref, g_ref):
    dinv = _dinv(d0_ref, d1_ref)
    h0 = jnp.dot(x_ref[...], we_ref[...], precision=lax.Precision.HIGHEST,
                 preferred_element_type=_f32) + be_ref[...]
    z = jnp.dot(h0, w1_ref[...], precision=lax.Precision.HIGHEST,
                preferred_element_type=_f32)
    g_ref[...] = dinv * z


def _step_body(p_ref, gin_ref, b_ref, w_ref, d0_ref, d1_ref, h_ref, g_ref):
    dinv = _dinv(d0_ref, d1_ref)
    t = p_ref[0] + p_ref[1] + gin_ref[...]
    h = jnp.maximum(dinv * t + b_ref[...], 0.0)
    h_ref[...] = h
    z = jnp.dot(h, w_ref[...], precision=lax.Precision.HIGHEST,
                preferred_element_type=_f32)
    g_ref[...] = dinv * z


_GRID = (N // BM,)
_spec_rows = pl.BlockSpec((BM, H), lambda i: (i, 0))
_spec_w = pl.BlockSpec((H, H), lambda i: (0, 0))
_spec_b = pl.BlockSpec((1, H), lambda i: (0, 0))
_spec_d = pl.BlockSpec((BM, 1), lambda i: (i, 0))
_spec_p = pl.BlockSpec((2, BM, H), lambda i: (0, i, 0))
_out_rows = jax.ShapeDtypeStruct((N, H), _f32)

_embed_call = pl.pallas_call(
    _embed_body, grid=_GRID,
    in_specs=[_spec_rows, _spec_w, _spec_b, _spec_w, _spec_d, _spec_d],
    out_specs=_spec_rows, out_shape=_out_rows)

_step_call = pl.pallas_call(
    _step_body, grid=_GRID,
    in_specs=[_spec_p, _spec_rows, _spec_b, _spec_w, _spec_d, _spec_d],
    out_specs=(_spec_rows, _spec_rows), out_shape=(_out_rows, _out_rows))


_PJ = np.arange(EPAD - E)
_PAD_SRC = np.asarray((_PJ * 37) % N, np.int32)
_PAD_DST = np.asarray(N + (_PJ % (ACC - N)), np.int32)


def kernel(x, edge_index, W_emb, b_emb, W1, b1, W2, b2, W3, b3):
    src = jnp.concatenate([edge_index[0], jnp.asarray(_PAD_SRC)]).reshape(EROWS, 128)
    dst = jnp.concatenate([edge_index[1], jnp.asarray(_PAD_DST)]).reshape(EROWS, 128)

    degdump = _deg_call(dst)
    d0 = lax.slice(degdump, (0,), (N,)).reshape(N, 1)
    d1 = lax.slice(degdump, (ACC,), (ACC + N,)).reshape(N, 1)

    g = _embed_call(x, W_emb, b_emb.reshape(1, H), W1, d0, d1)

    Ws = jnp.stack([W2, W3, W3])
    bs = jnp.stack([b1.reshape(1, H), b2.reshape(1, H), b3.reshape(1, H)])

    def body(l, carry):
        g, _ = carry
        p = _agg_call(g, src, dst)
        W = lax.dynamic_index_in_dim(Ws, l, 0, keepdims=False)
        b = lax.dynamic_index_in_dim(bs, l, 0, keepdims=False)
        hh, gg = _step_call(p, g, b, W, d0, d1)
        return (gg, hh)

    _, h = lax.fori_loop(0, 3, body, (g, jnp.zeros((N, H), _f32)))
    return h


# speedup vs baseline: 1.0102x; 1.0102x over previous
import jax
import jax.numpy as jnp
import numpy as np
from jax import lax
from jax.experimental import pallas as pl
from jax.experimental.pallas import tpu as pltpu
from jax.experimental.pallas import tpu_sc as plsc

N = 10000
E = 320000
H = 128
EPAD = 327680
EROWS = EPAD // H
RW = 80
ACC = 10240
STRIPE = ACC // 16
BM = 2000

_f32 = jnp.float32


def _deg_body(dst_hbm, deg_out, acc3, dst_v, ones_v, zero_v):
    c = lax.axis_index("c")
    s = lax.axis_index("s")
    wid = c * 16 + s

    def z640(j, carry):
        zero_v[pl.ds(j * 16, 16)] = jnp.zeros((16,), _f32)
        return carry
    lax.fori_loop(0, 40, z640, 0)
    pltpu.sync_copy(zero_v, acc3.at[pl.ds(s * 640, 640)])

    def o128(j, carry):
        ones_v[pl.ds(j * 16, 16)] = jnp.full((16,), 1.0, _f32)
        return carry
    lax.fori_loop(0, 8, o128, 0)

    pltpu.sync_copy(dst_hbm.at[pl.ds(wid * RW, RW)], dst_v)
    plsc.subcore_barrier()

    def scat(i, carry):
        pltpu.sync_copy(ones_v, acc3.at[dst_v.at[i]], add=True)
        return carry
    lax.fori_loop(0, RW, scat, 0)
    plsc.subcore_barrier()

    pltpu.sync_copy(acc3.at[pl.ds(s * 640, 640)],
                    deg_out.at[pl.ds(c * ACC + s * 640, 640)])


_deg_call = pl.kernel(
    _deg_body,
    out_type=jax.ShapeDtypeStruct((2 * ACC,), _f32),
    mesh=plsc.VectorSubcoreMesh(core_axis_name="c", subcore_axis_name="s"),
    scratch_types=[
        pltpu.VMEM_SHARED((ACC,), _f32),
        pltpu.VMEM((RW, 128), jnp.int32),
        pltpu.VMEM((128,), _f32),
        pltpu.VMEM((640,), _f32),
    ],
)


def _agg_body(g_hbm, src_hbm, dst_hbm, p_out, acc, src_v, dst_v, r0, s0, s1):
    c = lax.axis_index("c")
    s = lax.axis_index("s")
    wid = c * 16 + s

    def fill_zero(k, carry):
        i = k // 8
        j = lax.rem(k, 8)
        r0[i, pl.ds(j * 16, 16)] = jnp.zeros((16,), _f32)
        return carry
    lax.fori_loop(0, 1024, fill_zero, 0)

    for k in range(STRIPE // 128):
        pltpu.sync_copy(r0, acc.at[pl.ds(s * STRIPE + k * 128, 128)])
    plsc.subcore_barrier()

    pltpu.sync_copy(src_hbm.at[pl.ds(wid * RW, RW)], src_v.at[pl.ds(0, RW)])
    pltpu.sync_copy(src_hbm.at[pl.ds(wid * RW, 8)], src_v.at[pl.ds(RW, 8)])
    pltpu.sync_copy(dst_hbm.at[pl.ds(wid * RW, RW)], dst_v)

    ha = r0.at[pl.ds(0, 64)]
    hb = r0.at[pl.ds(64, 64)]
    pltpu.async_copy(g_hbm.at[src_v.at[0, pl.ds(0, 64)]], ha, s0)
    pltpu.async_copy(g_hbm.at[src_v.at[0, pl.ds(64, 64)]], hb, s1)

    def step(i, carry):
        pltpu.make_async_copy(g_hbm.at[src_v.at[i, pl.ds(0, 64)]], ha, s0).wait()
        pltpu.make_async_copy(g_hbm.at[src_v.at[i, pl.ds(64, 64)]], hb, s1).wait()
        pltpu.sync_copy(r0, acc.at[dst_v.at[i]], add=True)
        pltpu.async_copy(g_hbm.at[src_v.at[i + 1, pl.ds(0, 64)]], ha, s0)
        pltpu.async_copy(g_hbm.at[src_v.at[i + 1, pl.ds(64, 64)]], hb, s1)
        return carry
    lax.fori_loop(0, RW, step, 0)
    pltpu.make_async_copy(g_hbm.at[src_v.at[RW, pl.ds(0, 64)]], ha, s0).wait()
    pltpu.make_async_copy(g_hbm.at[src_v.at[RW, pl.ds(64, 64)]], hb, s1).wait()
    plsc.subcore_barrier()

    pltpu.sync_copy(acc.at[pl.ds(s * STRIPE, STRIPE)],
                    p_out.at[c, pl.ds(s * STRIPE, STRIPE)])


_agg_call = pl.kernel(
    _agg_body,
    out_type=jax.ShapeDtypeStruct((2, ACC, H), _f32),
    mesh=plsc.VectorSubcoreMesh(core_axis_name="c", subcore_axis_name="s"),
    scratch_types=[
        pltpu.VMEM_SHARED((ACC, H), _f32),
        pltpu.VMEM((RW + 8, 128), jnp.int32),
        pltpu.VMEM((RW, 128), jnp.int32),
        pltpu.VMEM((128, H), _f32),
        pltpu.SemaphoreType.DMA,
        pltpu.SemaphoreType.DMA,
    ],
)


def _dinv(d0_ref, d1_ref):
    return lax.rsqrt(d0_ref[...] + d1_ref[...] + 1.0)


def _embed_body(x_ref, we_ref, be_ref, w1_ref, d0_ref, d1_ref, g_ref):
    dinv = _dinv(d0_ref, d1_ref)
    h0 = jnp.dot(x_ref[...], we_ref[...], precision=lax.Precision.HIGHEST,
                 preferred_element_type=_f32) + be_ref[...]
    z = jnp.dot(h0, w1_ref[...], precision=lax.Precision.HIGHEST,
                preferred_element_type=_f32)
    g_ref[...] = dinv * z


def _step_body(p_ref, gin_ref, b_ref, w_ref, d0_ref, d1_ref, h_ref, g_ref):
    dinv = _dinv(d0_ref, d1_ref)
    t = p_ref[0] + p_ref[1] + gin_ref[...]
    h = jnp.maximum(dinv * t + b_ref[...], 0.0)
    h_ref[...] = h
    z = jnp.dot(h, w_ref[...], precision=lax.Precision.HIGHEST,
                preferred_element_type=_f32)
    g_ref[...] = dinv * z


_GRID = (N // BM,)
_spec_rows = pl.BlockSpec((BM, H), lambda i: (i, 0))
_spec_w = pl.BlockSpec((H, H), lambda i: (0, 0))
_spec_b = pl.BlockSpec((1, H), lambda i: (0, 0))
_spec_d = pl.BlockSpec((BM, 1), lambda i: (i, 0))
_spec_p = pl.BlockSpec((2, BM, H), lambda i: (0, i, 0))
_out_rows = jax.ShapeDtypeStruct((N, H), _f32)

_embed_call = pl.pallas_call(
    _embed_body, grid=_GRID,
    in_specs=[_spec_rows, _spec_w, _spec_b, _spec_w, _spec_d, _spec_d],
    out_specs=_spec_rows, out_shape=_out_rows)

_step_call = pl.pallas_call(
    _step_body, grid=_GRID,
    in_specs=[_spec_p, _spec_rows, _spec_b, _spec_w, _spec_d, _spec_d],
    out_specs=(_spec_rows, _spec_rows), out_shape=(_out_rows, _out_rows))


_PJ = np.arange(EPAD - E)
_PAD_SRC = np.asarray((_PJ * 37) % N, np.int32)
_PAD_DST = np.asarray(N + (_PJ % (ACC - N)), np.int32)


def kernel(x, edge_index, W_emb, b_emb, W1, b1, W2, b2, W3, b3):
    src = jnp.concatenate([edge_index[0], jnp.asarray(_PAD_SRC)]).reshape(EROWS, 128)
    dst = jnp.concatenate([edge_index[1], jnp.asarray(_PAD_DST)]).reshape(EROWS, 128)

    degdump = _deg_call(dst)
    d0 = lax.slice(degdump, (0,), (N,)).reshape(N, 1)
    d1 = lax.slice(degdump, (ACC,), (ACC + N,)).reshape(N, 1)

    g = _embed_call(x, W_emb, b_emb.reshape(1, H), W1, d0, d1)

    Ws = jnp.stack([W2, W3, W3])
    bs = jnp.stack([b1.reshape(1, H), b2.reshape(1, H), b3.reshape(1, H)])

    def body(l, carry):
        g, _ = carry
        p = _agg_call(g, src, dst)
        W = lax.dynamic_index_in_dim(Ws, l, 0, keepdims=False)
        b = lax.dynamic_index_in_dim(bs, l, 0, keepdims=False)
        hh, gg = _step_call(p, g, b, W, d0, d1)
        return (gg, hh)

    _, h = lax.fori_loop(0, 3, body, (g, jnp.zeros((N, H), _f32)))
    return h
